# baseline (speedup 1.0000x reference)
import functools

import jax
import jax.numpy as jnp
from jax import lax
from jax.experimental import pallas as pl
from jax.experimental.pallas import tpu as pltpu
from jax.experimental.pallas import tpu_sc as plsc

_B = 16384
_D = 64
_NW = 32
_BPW = _B // _NW
_CHUNK = 128
_NCHUNK = _BPW // _CHUNK

_MLP_BLK = 2048


def _gather_body(uidx_hbm, iidx_hbm, utab_hbm, itab_hbm, ue_hbm, ie_hbm,
                 idx_u, idx_i, rows_u, rows_i, sem):
    wid = lax.axis_index("s") * 2 + lax.axis_index("c")
    base = wid * _BPW
    pltpu.sync_copy(uidx_hbm.at[pl.ds(base, _BPW)], idx_u)
    pltpu.sync_copy(iidx_hbm.at[pl.ds(base, _BPW)], idx_i)
    cps = []
    for j in range(_NCHUNK):
        sl = pl.ds(j * _CHUNK, _CHUNK)
        cps.append(pltpu.async_copy(utab_hbm.at[idx_u.at[sl]], rows_u.at[sl], sem))
        cps.append(pltpu.async_copy(itab_hbm.at[idx_i.at[sl]], rows_i.at[sl], sem))
    for c in cps:
        c.wait()
    pltpu.sync_copy(rows_u, ue_hbm.at[pl.ds(base, _BPW)])
    pltpu.sync_copy(rows_i, ie_hbm.at[pl.ds(base, _BPW)])


@functools.cache
def _sc_gather():
    return pl.kernel(
        _gather_body,
        out_type=(
            jax.ShapeDtypeStruct((_B, _D), jnp.float32),
            jax.ShapeDtypeStruct((_B, _D), jnp.float32),
        ),
        mesh=plsc.VectorSubcoreMesh(core_axis_name="c", subcore_axis_name="s"),
        compiler_params=pltpu.CompilerParams(use_tc_tiling_on_sc=False),
        scratch_types=[
            pltpu.VMEM((_BPW,), jnp.int32),
            pltpu.VMEM((_BPW,), jnp.int32),
            pltpu.VMEM((_BPW, _D), jnp.float32),
            pltpu.VMEM((_BPW, _D), jnp.float32),
            pltpu.SemaphoreType.DMA,
        ],
    )


def _mlp_body(ue_ref, ie_ref, w0, b0, w1, b1, w2, b2, w3, b3, wo, bo, out_ref):
    hp = jnp.float32
    h = jnp.dot(ue_ref[...], w0[0:_D, :], preferred_element_type=hp)
    h = h + jnp.dot(ie_ref[...], w0[_D:2 * _D, :], preferred_element_type=hp)
    h = jnp.maximum(h + b0[...], 0.0)
    h = jnp.maximum(jnp.dot(h, w1[...], preferred_element_type=hp) + b1[...], 0.0)
    h = jnp.maximum(jnp.dot(h, w2[...], preferred_element_type=hp) + b2[...], 0.0)
    h = jnp.maximum(jnp.dot(h, w3[...], preferred_element_type=hp) + b3[...], 0.0)
    logits = jnp.sum(h * wo[...], axis=1) + bo[0, 0]
    out_ref[...] = 5.0 * jax.nn.sigmoid(logits)


def _mlp(ue, ie, W0, b0, W1, b1, W2, b2, W3, b3, Wo, bo):
    full = lambda shape: pl.BlockSpec(shape, lambda i: (0,) * len(shape))
    grid = _B // _MLP_BLK
    return pl.pallas_call(
        _mlp_body,
        grid=(grid,),
        in_specs=[
            pl.BlockSpec((_MLP_BLK, _D), lambda i: (i, 0)),
            pl.BlockSpec((_MLP_BLK, _D), lambda i: (i, 0)),
            full(W0.shape), full(b0.shape),
            full(W1.shape), full(b1.shape),
            full(W2.shape), full(b2.shape),
            full(W3.shape), full(b3.shape),
            full(Wo.shape), full(bo.shape),
        ],
        out_specs=pl.BlockSpec((_MLP_BLK,), lambda i: (i,)),
        out_shape=jax.ShapeDtypeStruct((_B,), jnp.float32),
    )(ue, ie, W0, b0, W1, b1, W2, b2, W3, b3, Wo, bo)


@jax.jit
def kernel(user_input, item_input, user_table, item_table,
           W0, b0, W1, b1, W2, b2, W3, b3, Wo, bo):
    ue, ie = _sc_gather()(user_input, item_input, user_table, item_table)
    b0r = b0.reshape(1, -1)
    b1r = b1.reshape(1, -1)
    b2r = b2.reshape(1, -1)
    b3r = b3.reshape(1, -1)
    wor = Wo.reshape(1, -1)
    bor = bo.reshape(1, 1)
    return _mlp(ue, ie, W0, b0r, W1, b1r, W2, b2r, W3, b3r, wor, bor)


# speedup vs baseline: 1.2178x; 1.2178x over previous
import functools

import jax
import jax.numpy as jnp
from jax import lax
from jax.experimental import pallas as pl
from jax.experimental.pallas import tpu as pltpu
from jax.experimental.pallas import tpu_sc as plsc

_B = 16384
_D = 64
_NW = 32
_BPW = _B // _NW
_CHUNK = 128
_NCHUNK = _BPW // _CHUNK

_MLP_BLK = 2048


def _gather_body(uidx_hbm, iidx_hbm, comb_hbm, gu_hbm, gi_hbm,
                 idx_u, idx_i, rows, sem):
    wid = lax.axis_index("s") * 2 + lax.axis_index("c")
    base = wid * _BPW
    pltpu.sync_copy(uidx_hbm.at[pl.ds(base, _BPW)], idx_u)
    pltpu.sync_copy(iidx_hbm.at[pl.ds(base, _BPW)], idx_i)
    for idx, out in ((idx_u, gu_hbm), (idx_i, gi_hbm)):
        cps = []
        for j in range(_NCHUNK):
            sl = pl.ds(j * _CHUNK, _CHUNK)
            cps.append(pltpu.async_copy(comb_hbm.at[idx.at[sl]], rows.at[sl], sem))
        for c in cps:
            c.wait()
        pltpu.sync_copy(rows, out.at[pl.ds(base, _BPW)])


@functools.cache
def _sc_gather():
    return pl.kernel(
        _gather_body,
        out_type=(
            jax.ShapeDtypeStruct((_B, 2 * _D), jnp.float32),
            jax.ShapeDtypeStruct((_B, 2 * _D), jnp.float32),
        ),
        mesh=plsc.VectorSubcoreMesh(core_axis_name="c", subcore_axis_name="s"),
        scratch_types=[
            pltpu.VMEM((_BPW,), jnp.int32),
            pltpu.VMEM((_BPW,), jnp.int32),
            pltpu.VMEM((_BPW, 2 * _D), jnp.float32),
            pltpu.SemaphoreType.DMA,
        ],
    )


def _mlp_body(gu_ref, gi_ref, w0u, w0i, b0, w1, b1, w2, b2, w3, b3,
              wo, bo, out_ref):
    hp = jnp.float32
    h = jnp.dot(gu_ref[...], w0u[...], preferred_element_type=hp)
    h = h + jnp.dot(gi_ref[...], w0i[...], preferred_element_type=hp)
    h = jnp.maximum(h + b0[...], 0.0)
    h = jnp.maximum(jnp.dot(h, w1[...], preferred_element_type=hp) + b1[...], 0.0)
    h = jnp.maximum(jnp.dot(h, w2[...], preferred_element_type=hp) + b2[...], 0.0)
    h = jnp.maximum(jnp.dot(h, w3[...], preferred_element_type=hp) + b3[...], 0.0)
    logits = jnp.sum(h * wo[...], axis=1) + bo[0, 0]
    out_ref[...] = 5.0 * jax.nn.sigmoid(logits)


def _mlp(gu, gi, w0u, w0i, b0, W1, b1, W2, b2, W3, b3, wo, bo):
    full = lambda shape: pl.BlockSpec(shape, lambda i: (0,) * len(shape))
    grid = _B // _MLP_BLK
    return pl.pallas_call(
        _mlp_body,
        grid=(grid,),
        in_specs=[
            pl.BlockSpec((_MLP_BLK, 2 * _D), lambda i: (i, 0)),
            pl.BlockSpec((_MLP_BLK, 2 * _D), lambda i: (i, 0)),
            full(w0u.shape), full(w0i.shape), full(b0.shape),
            full(W1.shape), full(b1.shape),
            full(W2.shape), full(b2.shape),
            full(W3.shape), full(b3.shape),
            full(wo.shape), full(bo.shape),
        ],
        out_specs=pl.BlockSpec((_MLP_BLK,), lambda i: (i,)),
        out_shape=jax.ShapeDtypeStruct((_B,), jnp.float32),
    )(gu, gi, w0u, w0i, b0, W1, b1, W2, b2, W3, b3, wo, bo)


@jax.jit
def kernel(user_input, item_input, user_table, item_table,
           W0, b0, W1, b1, W2, b2, W3, b3, Wo, bo):
    comb = jnp.concatenate([user_table, item_table], axis=1)
    gu, gi = _sc_gather()(user_input, item_input, comb)
    z = jnp.zeros((_D, W0.shape[1]), W0.dtype)
    w0u = jnp.concatenate([W0[:_D, :], z], axis=0)
    w0i = jnp.concatenate([z, W0[_D:, :]], axis=0)
    return _mlp(
        gu, gi, w0u, w0i, b0.reshape(1, -1),
        W1, b1.reshape(1, -1),
        W2, b2.reshape(1, -1),
        W3, b3.reshape(1, -1),
        Wo.reshape(1, -1), bo.reshape(1, 1),
    )


# speedup vs baseline: 1.6388x; 1.3457x over previous
import functools

import jax
import jax.numpy as jnp
from jax import lax
from jax.experimental import pallas as pl
from jax.experimental.pallas import tpu as pltpu
from jax.experimental.pallas import tpu_sc as plsc

_B = 16384
_D = 64
_NW = 32
_BPW = _B // _NW
_CHUNK = 128
_NCHUNK = _BPW // _CHUNK

_MLP_BLK = 2048
_TR_BLK = 2048
_NROWS = 1000000


def _transpose_body(tu_ref, ti_ref, out_ref):
    out_ref[...] = jnp.concatenate(
        [jnp.swapaxes(tu_ref[...], 0, 1), jnp.swapaxes(ti_ref[...], 0, 1)],
        axis=1)


def _build_combined(tu, ti):
    grid = (_NROWS + _TR_BLK - 1) // _TR_BLK
    return pl.pallas_call(
        _transpose_body,
        grid=(grid,),
        in_specs=[
            pl.BlockSpec((_D, _TR_BLK), lambda i: (0, i)),
            pl.BlockSpec((_D, _TR_BLK), lambda i: (0, i)),
        ],
        out_specs=pl.BlockSpec((_TR_BLK, 2 * _D), lambda i: (i, 0)),
        out_shape=jax.ShapeDtypeStruct((_NROWS, 2 * _D), jnp.float32),
    )(tu, ti)


def _gather_body(uidx_hbm, iidx_hbm, comb_hbm, gu_hbm, gi_hbm,
                 idx_u, idx_i, rows, sem):
    wid = lax.axis_index("s") * 2 + lax.axis_index("c")
    base = wid * _BPW
    pltpu.sync_copy(uidx_hbm.at[pl.ds(base, _BPW)], idx_u)
    pltpu.sync_copy(iidx_hbm.at[pl.ds(base, _BPW)], idx_i)
    for idx, out in ((idx_u, gu_hbm), (idx_i, gi_hbm)):
        cps = []
        for j in range(_NCHUNK):
            sl = pl.ds(j * _CHUNK, _CHUNK)
            cps.append(pltpu.async_copy(comb_hbm.at[idx.at[sl]], rows.at[sl], sem))
        for c in cps:
            c.wait()
        pltpu.sync_copy(rows, out.at[pl.ds(base, _BPW)])


@functools.cache
def _sc_gather():
    return pl.kernel(
        _gather_body,
        out_type=(
            jax.ShapeDtypeStruct((_B, 2 * _D), jnp.float32),
            jax.ShapeDtypeStruct((_B, 2 * _D), jnp.float32),
        ),
        mesh=plsc.VectorSubcoreMesh(core_axis_name="c", subcore_axis_name="s"),
        scratch_types=[
            pltpu.VMEM((_BPW,), jnp.int32),
            pltpu.VMEM((_BPW,), jnp.int32),
            pltpu.VMEM((_BPW, 2 * _D), jnp.float32),
            pltpu.SemaphoreType.DMA,
        ],
    )


def _mlp_body(gu_ref, gi_ref, w0u, w0i, b0, w1, b1, w2, b2, w3, b3,
              wo, bo, out_ref):
    hp = jnp.float32
    h = jnp.dot(gu_ref[...], w0u[...], preferred_element_type=hp)
    h = h + jnp.dot(gi_ref[...], w0i[...], preferred_element_type=hp)
    h = jnp.maximum(h + b0[...], 0.0)
    h = jnp.maximum(jnp.dot(h, w1[...], preferred_element_type=hp) + b1[...], 0.0)
    h = jnp.maximum(jnp.dot(h, w2[...], preferred_element_type=hp) + b2[...], 0.0)
    h = jnp.maximum(jnp.dot(h, w3[...], preferred_element_type=hp) + b3[...], 0.0)
    logits = jnp.sum(h * wo[...], axis=1) + bo[0, 0]
    out_ref[...] = 5.0 * jax.nn.sigmoid(logits)


def _mlp(gu, gi, w0u, w0i, b0, W1, b1, W2, b2, W3, b3, wo, bo):
    full = lambda shape: pl.BlockSpec(shape, lambda i: (0,) * len(shape))
    grid = _B // _MLP_BLK
    return pl.pallas_call(
        _mlp_body,
        grid=(grid,),
        in_specs=[
            pl.BlockSpec((_MLP_BLK, 2 * _D), lambda i: (i, 0)),
            pl.BlockSpec((_MLP_BLK, 2 * _D), lambda i: (i, 0)),
            full(w0u.shape), full(w0i.shape), full(b0.shape),
            full(W1.shape), full(b1.shape),
            full(W2.shape), full(b2.shape),
            full(W3.shape), full(b3.shape),
            full(wo.shape), full(bo.shape),
        ],
        out_specs=pl.BlockSpec((_MLP_BLK,), lambda i: (i,)),
        out_shape=jax.ShapeDtypeStruct((_B,), jnp.float32),
    )(gu, gi, w0u, w0i, b0, W1, b1, W2, b2, W3, b3, wo, bo)


@jax.jit
def kernel(user_input, item_input, user_table, item_table,
           W0, b0, W1, b1, W2, b2, W3, b3, Wo, bo):
    comb = _build_combined(user_table.T, item_table.T)
    gu, gi = _sc_gather()(user_input, item_input, comb)
    z = jnp.zeros((_D, W0.shape[1]), W0.dtype)
    w0u = jnp.concatenate([W0[:_D, :], z], axis=0)
    w0i = jnp.concatenate([z, W0[_D:, :]], axis=0)
    return _mlp(
        gu, gi, w0u, w0i, b0.reshape(1, -1),
        W1, b1.reshape(1, -1),
        W2, b2.reshape(1, -1),
        W3, b3.reshape(1, -1),
        Wo.reshape(1, -1), bo.reshape(1, 1),
    )


# speedup vs baseline: 2.3186x; 1.4148x over previous
import functools

import jax
import jax.numpy as jnp
from jax import lax
from jax.experimental import pallas as pl
from jax.experimental.pallas import tpu as pltpu
from jax.experimental.pallas import tpu_sc as plsc

_B = 16384
_D = 64
_NW = 32
_BPW = _B // _NW
_CHUNK = 128
_NCHUNK = _BPW // _CHUNK

_MLP_BLK = 2048
_TR_BLK = 8192
_NROWS = 1000000


def _transpose_body(tu_ref, ti_ref, out_ref):
    out_ref[...] = jnp.concatenate(
        [jnp.swapaxes(tu_ref[...], 0, 1), jnp.swapaxes(ti_ref[...], 0, 1)],
        axis=1)


def _build_combined(tu, ti):
    grid = (_NROWS + _TR_BLK - 1) // _TR_BLK
    return pl.pallas_call(
        _transpose_body,
        grid=(grid,),
        in_specs=[
            pl.BlockSpec((_D, _TR_BLK), lambda i: (0, i)),
            pl.BlockSpec((_D, _TR_BLK), lambda i: (0, i)),
        ],
        out_specs=pl.BlockSpec((_TR_BLK, 2 * _D), lambda i: (i, 0)),
        out_shape=jax.ShapeDtypeStruct((_NROWS, 2 * _D), jnp.float32),
    )(tu, ti)


def _gather_body(uidx_hbm, iidx_hbm, comb_hbm, gu_hbm, gi_hbm,
                 idx_u, idx_i, rows, sem):
    wid = lax.axis_index("s") * 2 + lax.axis_index("c")
    base = wid * _BPW
    pltpu.sync_copy(uidx_hbm.at[pl.ds(base, _BPW)], idx_u)
    pltpu.sync_copy(iidx_hbm.at[pl.ds(base, _BPW)], idx_i)
    for idx, out in ((idx_u, gu_hbm), (idx_i, gi_hbm)):
        cps = []
        for j in range(_NCHUNK):
            sl = pl.ds(j * _CHUNK, _CHUNK)
            cps.append(pltpu.async_copy(comb_hbm.at[idx.at[sl]], rows.at[sl], sem))
        for c in cps:
            c.wait()
        pltpu.sync_copy(rows, out.at[pl.ds(base, _BPW)])


@functools.cache
def _sc_gather():
    return pl.kernel(
        _gather_body,
        out_type=(
            jax.ShapeDtypeStruct((_B, 2 * _D), jnp.float32),
            jax.ShapeDtypeStruct((_B, 2 * _D), jnp.float32),
        ),
        mesh=plsc.VectorSubcoreMesh(core_axis_name="c", subcore_axis_name="s"),
        scratch_types=[
            pltpu.VMEM((_BPW,), jnp.int32),
            pltpu.VMEM((_BPW,), jnp.int32),
            pltpu.VMEM((_BPW, 2 * _D), jnp.float32),
            pltpu.SemaphoreType.DMA,
        ],
    )


def _mlp_body(gu_ref, gi_ref, w0u, w0i, b0, w1, b1, w2, b2, w3, b3,
              wo, bo, out_ref):
    hp = jnp.float32
    h = jnp.dot(gu_ref[...], w0u[...], preferred_element_type=hp)
    h = h + jnp.dot(gi_ref[...], w0i[...], preferred_element_type=hp)
    h = jnp.maximum(h + b0[...], 0.0)
    h = jnp.maximum(jnp.dot(h, w1[...], preferred_element_type=hp) + b1[...], 0.0)
    h = jnp.maximum(jnp.dot(h, w2[...], preferred_element_type=hp) + b2[...], 0.0)
    h = jnp.maximum(jnp.dot(h, w3[...], preferred_element_type=hp) + b3[...], 0.0)
    logits = jnp.sum(h * wo[...], axis=1) + bo[0, 0]
    out_ref[...] = 5.0 * jax.nn.sigmoid(logits)


def _mlp(gu, gi, w0u, w0i, b0, W1, b1, W2, b2, W3, b3, wo, bo):
    full = lambda shape: pl.BlockSpec(shape, lambda i: (0,) * len(shape))
    grid = _B // _MLP_BLK
    return pl.pallas_call(
        _mlp_body,
        grid=(grid,),
        in_specs=[
            pl.BlockSpec((_MLP_BLK, 2 * _D), lambda i: (i, 0)),
            pl.BlockSpec((_MLP_BLK, 2 * _D), lambda i: (i, 0)),
            full(w0u.shape), full(w0i.shape), full(b0.shape),
            full(W1.shape), full(b1.shape),
            full(W2.shape), full(b2.shape),
            full(W3.shape), full(b3.shape),
            full(wo.shape), full(bo.shape),
        ],
        out_specs=pl.BlockSpec((_MLP_BLK,), lambda i: (i,)),
        out_shape=jax.ShapeDtypeStruct((_B,), jnp.float32),
    )(gu, gi, w0u, w0i, b0, W1, b1, W2, b2, W3, b3, wo, bo)


@jax.jit
def kernel(user_input, item_input, user_table, item_table,
           W0, b0, W1, b1, W2, b2, W3, b3, Wo, bo):
    comb = _build_combined(user_table.T, item_table.T)
    gu, gi = _sc_gather()(user_input, item_input, comb)
    z = jnp.zeros((_D, W0.shape[1]), W0.dtype)
    w0u = jnp.concatenate([W0[:_D, :], z], axis=0)
    w0i = jnp.concatenate([z, W0[_D:, :]], axis=0)
    return _mlp(
        gu, gi, w0u, w0i, b0.reshape(1, -1),
        W1, b1.reshape(1, -1),
        W2, b2.reshape(1, -1),
        W3, b3.reshape(1, -1),
        Wo.reshape(1, -1), bo.reshape(1, 1),
    )


# speedup vs baseline: 2.4614x; 1.0616x over previous
import functools

import jax
import jax.numpy as jnp
from jax import lax
from jax.experimental import pallas as pl
from jax.experimental.pallas import tpu as pltpu
from jax.experimental.pallas import tpu_sc as plsc

_B = 16384
_D = 64
_NW = 32
_BPW = _B // _NW
_CHUNK = 128
_NCHUNK = _BPW // _CHUNK

_MLP_BLK = 2048
_TR_BLK = 16384
_NROWS = 1000000


def _transpose_body(tu_ref, ti_ref, out_ref):
    out_ref[...] = jnp.concatenate(
        [jnp.swapaxes(tu_ref[...], 0, 1), jnp.swapaxes(ti_ref[...], 0, 1)],
        axis=1)


def _build_combined(tu, ti):
    grid = (_NROWS + _TR_BLK - 1) // _TR_BLK
    return pl.pallas_call(
        _transpose_body,
        grid=(grid,),
        in_specs=[
            pl.BlockSpec((_D, _TR_BLK), lambda i: (0, i)),
            pl.BlockSpec((_D, _TR_BLK), lambda i: (0, i)),
        ],
        out_specs=pl.BlockSpec((_TR_BLK, 2 * _D), lambda i: (i, 0)),
        out_shape=jax.ShapeDtypeStruct((_NROWS, 2 * _D), jnp.float32),
    )(tu, ti)


def _gather_body(uidx_hbm, iidx_hbm, comb_hbm, gu_hbm, gi_hbm,
                 idx_u, idx_i, rows, sem):
    wid = lax.axis_index("s") * 2 + lax.axis_index("c")
    base = wid * _BPW
    pltpu.sync_copy(uidx_hbm.at[pl.ds(base, _BPW)], idx_u)
    pltpu.sync_copy(iidx_hbm.at[pl.ds(base, _BPW)], idx_i)
    for idx, out in ((idx_u, gu_hbm), (idx_i, gi_hbm)):
        cps = []
        for j in range(_NCHUNK):
            sl = pl.ds(j * _CHUNK, _CHUNK)
            cps.append(pltpu.async_copy(comb_hbm.at[idx.at[sl]], rows.at[sl], sem))
        for c in cps:
            c.wait()
        pltpu.sync_copy(rows, out.at[pl.ds(base, _BPW)])


@functools.cache
def _sc_gather():
    return pl.kernel(
        _gather_body,
        out_type=(
            jax.ShapeDtypeStruct((_B, 2 * _D), jnp.float32),
            jax.ShapeDtypeStruct((_B, 2 * _D), jnp.float32),
        ),
        mesh=plsc.VectorSubcoreMesh(core_axis_name="c", subcore_axis_name="s"),
        scratch_types=[
            pltpu.VMEM((_BPW,), jnp.int32),
            pltpu.VMEM((_BPW,), jnp.int32),
            pltpu.VMEM((_BPW, 2 * _D), jnp.float32),
            pltpu.SemaphoreType.DMA,
        ],
    )


def _mlp_body(gu_ref, gi_ref, w0u, w0i, b0, w1, b1, w2, b2, w3, b3,
              wo, bo, out_ref):
    hp = jnp.float32
    h = jnp.dot(gu_ref[...], w0u[...], preferred_element_type=hp)
    h = h + jnp.dot(gi_ref[...], w0i[...], preferred_element_type=hp)
    h = jnp.maximum(h + b0[...], 0.0)
    h = jnp.maximum(jnp.dot(h, w1[...], preferred_element_type=hp) + b1[...], 0.0)
    h = jnp.maximum(jnp.dot(h, w2[...], preferred_element_type=hp) + b2[...], 0.0)
    h = jnp.maximum(jnp.dot(h, w3[...], preferred_element_type=hp) + b3[...], 0.0)
    logits = jnp.sum(h * wo[...], axis=1) + bo[0, 0]
    out_ref[...] = 5.0 * jax.nn.sigmoid(logits)


def _mlp(gu, gi, w0u, w0i, b0, W1, b1, W2, b2, W3, b3, wo, bo):
    full = lambda shape: pl.BlockSpec(shape, lambda i: (0,) * len(shape))
    grid = _B // _MLP_BLK
    return pl.pallas_call(
        _mlp_body,
        grid=(grid,),
        in_specs=[
            pl.BlockSpec((_MLP_BLK, 2 * _D), lambda i: (i, 0)),
            pl.BlockSpec((_MLP_BLK, 2 * _D), lambda i: (i, 0)),
            full(w0u.shape), full(w0i.shape), full(b0.shape),
            full(W1.shape), full(b1.shape),
            full(W2.shape), full(b2.shape),
            full(W3.shape), full(b3.shape),
            full(wo.shape), full(bo.shape),
        ],
        out_specs=pl.BlockSpec((_MLP_BLK,), lambda i: (i,)),
        out_shape=jax.ShapeDtypeStruct((_B,), jnp.float32),
    )(gu, gi, w0u, w0i, b0, W1, b1, W2, b2, W3, b3, wo, bo)


@jax.jit
def kernel(user_input, item_input, user_table, item_table,
           W0, b0, W1, b1, W2, b2, W3, b3, Wo, bo):
    comb = _build_combined(user_table.T, item_table.T)
    gu, gi = _sc_gather()(user_input, item_input, comb)
    z = jnp.zeros((_D, W0.shape[1]), W0.dtype)
    w0u = jnp.concatenate([W0[:_D, :], z], axis=0)
    w0i = jnp.concatenate([z, W0[_D:, :]], axis=0)
    return _mlp(
        gu, gi, w0u, w0i, b0.reshape(1, -1),
        W1, b1.reshape(1, -1),
        W2, b2.reshape(1, -1),
        W3, b3.reshape(1, -1),
        Wo.reshape(1, -1), bo.reshape(1, 1),
    )


# speedup vs baseline: 3.0227x; 1.2280x over previous
import functools

import jax
import jax.numpy as jnp
from jax import lax
from jax.experimental import pallas as pl
from jax.experimental.pallas import tpu as pltpu
from jax.experimental.pallas import tpu_sc as plsc

_B = 16384
_D = 64
_NW = 32
_BPW = _B // _NW
_CHUNK = 128
_NCHUNK = _BPW // _CHUNK

_MLP_BLK = 2048
_TR_BLK = 16384
_NROWS = 1000000


def _transpose_body(tu_ref, ti_ref, out_ref):
    tb = jnp.swapaxes(tu_ref[...].astype(jnp.bfloat16), 0, 1)
    ib = jnp.swapaxes(ti_ref[...].astype(jnp.bfloat16), 0, 1)
    out_ref[:, 0:_D] = tb.astype(jnp.float32)
    out_ref[:, _D:2 * _D] = ib.astype(jnp.float32)


def _build_combined(tu, ti):
    grid = (_NROWS + _TR_BLK - 1) // _TR_BLK
    return pl.pallas_call(
        _transpose_body,
        grid=(grid,),
        in_specs=[
            pl.BlockSpec((_D, _TR_BLK), lambda i: (0, i)),
            pl.BlockSpec((_D, _TR_BLK), lambda i: (0, i)),
        ],
        out_specs=pl.BlockSpec((_TR_BLK, 2 * _D), lambda i: (i, 0)),
        out_shape=jax.ShapeDtypeStruct((_NROWS, 2 * _D), jnp.float32),
    )(tu, ti)


def _gather_body(uidx_hbm, iidx_hbm, comb_hbm, gu_hbm, gi_hbm,
                 idx_u, idx_i, rows, sem):
    wid = lax.axis_index("s") * 2 + lax.axis_index("c")
    base = wid * _BPW
    pltpu.sync_copy(uidx_hbm.at[pl.ds(base, _BPW)], idx_u)
    pltpu.sync_copy(iidx_hbm.at[pl.ds(base, _BPW)], idx_i)
    for idx, out in ((idx_u, gu_hbm), (idx_i, gi_hbm)):
        cps = []
        for j in range(_NCHUNK):
            sl = pl.ds(j * _CHUNK, _CHUNK)
            cps.append(pltpu.async_copy(comb_hbm.at[idx.at[sl]], rows.at[sl], sem))
        for c in cps:
            c.wait()
        pltpu.sync_copy(rows, out.at[pl.ds(base, _BPW)])


@functools.cache
def _sc_gather():
    return pl.kernel(
        _gather_body,
        out_type=(
            jax.ShapeDtypeStruct((_B, 2 * _D), jnp.float32),
            jax.ShapeDtypeStruct((_B, 2 * _D), jnp.float32),
        ),
        mesh=plsc.VectorSubcoreMesh(core_axis_name="c", subcore_axis_name="s"),
        scratch_types=[
            pltpu.VMEM((_BPW,), jnp.int32),
            pltpu.VMEM((_BPW,), jnp.int32),
            pltpu.VMEM((_BPW, 2 * _D), jnp.float32),
            pltpu.SemaphoreType.DMA,
        ],
    )


def _mlp_body(gu_ref, gi_ref, w0u, w0i, b0, w1, b1, w2, b2, w3, b3,
              wo, bo, out_ref):
    hp = jnp.float32
    h = jnp.dot(gu_ref[...], w0u[...], preferred_element_type=hp)
    h = h + jnp.dot(gi_ref[...], w0i[...], preferred_element_type=hp)
    h = jnp.maximum(h + b0[...], 0.0)
    h = jnp.maximum(jnp.dot(h, w1[...], preferred_element_type=hp) + b1[...], 0.0)
    h = jnp.maximum(jnp.dot(h, w2[...], preferred_element_type=hp) + b2[...], 0.0)
    h = jnp.maximum(jnp.dot(h, w3[...], preferred_element_type=hp) + b3[...], 0.0)
    logits = jnp.sum(h * wo[...], axis=1) + bo[0, 0]
    out_ref[...] = 5.0 * jax.nn.sigmoid(logits)


def _mlp(gu, gi, w0u, w0i, b0, W1, b1, W2, b2, W3, b3, wo, bo):
    full = lambda shape: pl.BlockSpec(shape, lambda i: (0,) * len(shape))
    grid = _B // _MLP_BLK
    return pl.pallas_call(
        _mlp_body,
        grid=(grid,),
        in_specs=[
            pl.BlockSpec((_MLP_BLK, 2 * _D), lambda i: (i, 0)),
            pl.BlockSpec((_MLP_BLK, 2 * _D), lambda i: (i, 0)),
            full(w0u.shape), full(w0i.shape), full(b0.shape),
            full(W1.shape), full(b1.shape),
            full(W2.shape), full(b2.shape),
            full(W3.shape), full(b3.shape),
            full(wo.shape), full(bo.shape),
        ],
        out_specs=pl.BlockSpec((_MLP_BLK,), lambda i: (i,)),
        out_shape=jax.ShapeDtypeStruct((_B,), jnp.float32),
    )(gu, gi, w0u, w0i, b0, W1, b1, W2, b2, W3, b3, wo, bo)


@jax.jit
def kernel(user_input, item_input, user_table, item_table,
           W0, b0, W1, b1, W2, b2, W3, b3, Wo, bo):
    comb = _build_combined(user_table.T, item_table.T)
    gu, gi = _sc_gather()(user_input, item_input, comb)
    z = jnp.zeros((_D, W0.shape[1]), W0.dtype)
    w0u = jnp.concatenate([W0[:_D, :], z], axis=0)
    w0i = jnp.concatenate([z, W0[_D:, :]], axis=0)
    return _mlp(
        gu, gi, w0u, w0i, b0.reshape(1, -1),
        W1, b1.reshape(1, -1),
        W2, b2.reshape(1, -1),
        W3, b3.reshape(1, -1),
        Wo.reshape(1, -1), bo.reshape(1, 1),
    )
